# baseline (speedup 1.0000x reference)
import functools

import jax
import jax.numpy as jnp
from jax import lax
from jax.experimental import pallas as pl
from jax.experimental.pallas import tpu as pltpu
from jax.experimental.pallas import tpu_sc as plsc

N_USERS = 2000
N_ITEMS = 4000
N_NODES = 10000
EMB = 128
RELD = 64
E = 320000
N_LAYERS = 2

NC = 2
NS = 16
NW = NC * NS
EPW = E // NW
K = 80
NCHUNK = EPW // K
SLAB = 624
LAST = N_NODES - 15 * SLAB
NVEC = EMB // 16

_MESH = plsc.VectorSubcoreMesh(
    core_axis_name="c", subcore_axis_name="s", num_cores=NC, num_subcores=NS)


@functools.partial(
    pl.kernel,
    out_type=jax.ShapeDtypeStruct((NC, N_NODES, EMB), jnp.float32),
    mesh=_MESH,
    scratch_types=[
        pltpu.VMEM((EPW,), jnp.int32),
        pltpu.VMEM((NCHUNK, K), jnp.int32),
        pltpu.VMEM((K,), jnp.float32),
        pltpu.VMEM((K,), jnp.float32),
        pltpu.VMEM((K, EMB), jnp.float32),
        pltpu.VMEM((K, EMB), jnp.float32),
        pltpu.VMEM_SHARED((N_NODES, EMB), jnp.float32),
        pltpu.SemaphoreType.DMA,
        pltpu.SemaphoreType.DMA,
        pltpu.SemaphoreType.DMA,
        pltpu.SemaphoreType.DMA,
        pltpu.SemaphoreType.DMA,
        pltpu.SemaphoreType.DMA,
    ],
)
def _spmm(ego_hbm, col_hbm, row_hbm, val_hbm, zero_hbm, out_hbm,
          col_all, row_all, val0, val1, rows0, rows1, acc,
          vsem0, vsem1, gsem0, gsem1, ssem0, ssem1):
    c = lax.axis_index("c")
    s = lax.axis_index("s")
    wid = s * NC + c
    ebase = wid * EPW

    pltpu.sync_copy(col_hbm.at[pl.ds(ebase, EPW)], col_all)
    pltpu.sync_copy(row_hbm.at[wid], row_all)

    @pl.when(s < 15)
    def _():
        pltpu.sync_copy(zero_hbm.at[pl.ds(0, SLAB)],
                        acc.at[pl.ds(s * SLAB, SLAB)])

    @pl.when(s == 15)
    def _():
        pltpu.sync_copy(zero_hbm, acc.at[pl.ds(15 * SLAB, LAST)])

    plsc.subcore_barrier()

    def idx_issue(i, valb, vsem):
        pltpu.async_copy(val_hbm.at[pl.ds(ebase + i * K, K)], valb, vsem)

    def idx_wait(valb, vsem):
        pltpu.make_async_copy(val_hbm.at[pl.ds(0, K)], valb, vsem).wait()

    def gather_issue(i, buf, gsem):
        pltpu.async_copy(ego_hbm.at[col_all.at[pl.ds(i * K, K)]], buf, gsem)

    def gather_wait(buf, gsem):
        pltpu.make_async_copy(ego_hbm.at[pl.ds(0, K)], buf, gsem).wait()

    def scat_issue(buf, i, ssem):
        pltpu.async_copy(buf, acc.at[row_all.at[i]], ssem, add=True)

    def scat_wait(buf, ssem):
        pltpu.make_async_copy(buf, acc.at[pl.ds(0, K)], ssem).wait()

    def scale(buf, valb):
        def s16(jj, c2):
            off = pl.multiple_of(jj * 16, 16)
            vals16 = valb[pl.ds(off, 16)]
            for l in range(16):
                j = off + l
                v = vals16[l]
                for g in range(NVEC):
                    sl = pl.ds(g * 16, 16)
                    buf[j, sl] = buf[j, sl] * v
            return c2

        lax.fori_loop(0, K // 16, s16, 0)

    idx_issue(0, val0, vsem0)
    gather_issue(0, rows0, gsem0)
    idx_issue(1, val1, vsem1)
    gather_issue(1, rows1, gsem1)

    def body(t, carry):
        a = 2 * t
        idx_wait(val0, vsem0)
        gather_wait(rows0, gsem0)
        scale(rows0, val0)
        scat_issue(rows0, a, ssem0)
        idx_wait(val1, vsem1)
        gather_wait(rows1, gsem1)
        scale(rows1, val1)
        scat_issue(rows1, a + 1, ssem1)
        scat_wait(rows0, ssem0)
        idx_issue(a + 2, val0, vsem0)
        gather_issue(a + 2, rows0, gsem0)
        nb = jnp.minimum(a + 3, NCHUNK - 1)
        scat_wait(rows1, ssem1)
        idx_issue(nb, val1, vsem1)
        gather_issue(nb, rows1, gsem1)
        return carry

    lax.fori_loop(0, (NCHUNK - 1) // 2, body, 0)

    idx_wait(val0, vsem0)
    gather_wait(rows0, gsem0)
    scale(rows0, val0)
    scat_issue(rows0, NCHUNK - 1, ssem0)
    idx_wait(val1, vsem1)
    gather_wait(rows1, gsem1)
    scat_wait(rows0, ssem0)
    plsc.subcore_barrier()

    @pl.when(s < 15)
    def _():
        pltpu.sync_copy(acc.at[pl.ds(s * SLAB, SLAB)],
                        out_hbm.at[c, pl.ds(s * SLAB, SLAB)])

    @pl.when(s == 15)
    def _():
        pltpu.sync_copy(acc.at[pl.ds(15 * SLAB, LAST)],
                        out_hbm.at[c, pl.ds(15 * SLAB, LAST)])


def _dense_body(ego_ref, np_ref, wr_ref, re_ref, w1t_ref, w3t_ref, out_ref):
    ego = ego_ref[...]
    neigh = np_ref[0] + np_ref[1]
    wr = wr_ref[...]
    h = jnp.dot(ego, wr, preferred_element_type=jnp.float32)
    t = jnp.dot(neigh, wr, preferred_element_type=jnp.float32)
    score = jnp.sum(t * jnp.tanh(h + re_ref[...]), axis=1, keepdims=True)
    m = jnp.max(score)
    ex = jnp.exp(score - m)
    neigh = neigh * (ex / jnp.sum(ex))
    a = jnp.dot(ego + neigh, w1t_ref[...], preferred_element_type=jnp.float32)
    b = jnp.dot(ego * neigh, w3t_ref[...], preferred_element_type=jnp.float32)
    out_ref[...] = (jnp.where(a >= 0, a, 0.2 * a)
                    + jnp.where(b >= 0, b, 0.2 * b))


def _dense(ego, neigh_parts, wr, re_, w1t, w3t):
    return pl.pallas_call(
        _dense_body,
        out_shape=jax.ShapeDtypeStruct((N_NODES, EMB), jnp.float32),
    )(ego, neigh_parts, wr, re_, w1t, w3t)


NUI = N_USERS + N_ITEMS


def _dense2_body(ent_ref, ego_ref, np_ref, wr_ref, re_ref, w1t_ref, w3t_ref,
                 user_ref, item_ref):
    ego = ego_ref[...]
    neigh = np_ref[0] + np_ref[1]
    wr = wr_ref[...]
    h = jnp.dot(ego, wr, preferred_element_type=jnp.float32)
    t = jnp.dot(neigh, wr, preferred_element_type=jnp.float32)
    score = jnp.sum(t * jnp.tanh(h + re_ref[...]), axis=1, keepdims=True)
    m = jnp.max(score)
    ex = jnp.exp(score - m)
    attn = ex / jnp.sum(ex)
    egon = ego[:NUI]
    neighn = neigh[:NUI] * attn[:NUI]
    a = jnp.dot(egon + neighn, w1t_ref[...], preferred_element_type=jnp.float32)
    b = jnp.dot(egon * neighn, w3t_ref[...], preferred_element_type=jnp.float32)
    res = (jnp.where(a >= 0, a, 0.2 * a)
           + jnp.where(b >= 0, b, 0.2 * b))
    user_ref[:, 0:EMB] = ent_ref[0:N_USERS]
    user_ref[:, EMB:2 * EMB] = ego[0:N_USERS]
    user_ref[:, 2 * EMB:] = res[0:N_USERS]
    item_ref[:, 0:EMB] = ent_ref[N_USERS:NUI]
    item_ref[:, EMB:2 * EMB] = ego[N_USERS:NUI]
    item_ref[:, 2 * EMB:] = res[N_USERS:NUI]


def _dense2(ent_emb, ego, neigh_parts, wr, re_, w1t, w3t):
    return pl.pallas_call(
        _dense2_body,
        out_shape=(jax.ShapeDtypeStruct((N_USERS, 3 * EMB), jnp.float32),
                   jax.ShapeDtypeStruct((N_ITEMS, 3 * EMB), jnp.float32)),
    )(ent_emb, ego, neigh_parts, wr, re_, w1t, w3t)


def kernel(ent_emb, rel_emb, rel_proj, W1, W3, edge_val, edge_row, edge_col):
    wr = rel_proj[0].reshape(EMB, RELD)
    re_ = rel_emb[0].reshape(1, RELD)
    w1t = W1.T
    w3t = W3.T
    zeros = jnp.zeros((LAST, EMB), jnp.float32)
    row3 = edge_row.reshape(NW, NCHUNK, K)
    parts = _spmm(ent_emb, edge_col, row3, edge_val, zeros)
    ego1 = _dense(ent_emb, parts, wr, re_, w1t, w3t)
    parts = _spmm(ego1, edge_col, row3, edge_val, zeros)
    return _dense2(ent_emb, ego1, parts, wr, re_, w1t, w3t)


# speedup vs baseline: 1.0097x; 1.0097x over previous
import functools

import jax
import jax.numpy as jnp
from jax import lax
from jax.experimental import pallas as pl
from jax.experimental.pallas import tpu as pltpu
from jax.experimental.pallas import tpu_sc as plsc

N_USERS = 2000
N_ITEMS = 4000
N_NODES = 10000
EMB = 128
RELD = 64
E = 320000
N_LAYERS = 2

NC = 2
NS = 16
NW = NC * NS
EPW = E // NW
K = 80
NCHUNK = EPW // K
SLAB = 624
LAST = N_NODES - 15 * SLAB
NVEC = EMB // 16

_MESH = plsc.VectorSubcoreMesh(
    core_axis_name="c", subcore_axis_name="s", num_cores=NC, num_subcores=NS)


@functools.partial(
    pl.kernel,
    out_type=jax.ShapeDtypeStruct((NC, N_NODES, EMB), jnp.float32),
    mesh=_MESH,
    scratch_types=[
        pltpu.VMEM((EPW,), jnp.int32),
        pltpu.VMEM((K,), jnp.int32),
        pltpu.VMEM((K,), jnp.int32),
        pltpu.VMEM((K,), jnp.float32),
        pltpu.VMEM((K,), jnp.float32),
        pltpu.VMEM((K, EMB), jnp.float32),
        pltpu.VMEM((K, EMB), jnp.float32),
        pltpu.VMEM_SHARED((N_NODES, EMB), jnp.float32),
        pltpu.SemaphoreType.DMA,
        pltpu.SemaphoreType.DMA,
        pltpu.SemaphoreType.DMA,
        pltpu.SemaphoreType.DMA,
        pltpu.SemaphoreType.DMA,
        pltpu.SemaphoreType.DMA,
        pltpu.SemaphoreType.DMA,
        pltpu.SemaphoreType.DMA,
    ],
)
def _spmm(ego_hbm, col_hbm, row_hbm, val_hbm, zero_hbm, out_hbm,
          col_all, row0, row1, val0, val1, rows0, rows1, acc,
          rsem0, rsem1, vsem0, vsem1, gsem0, gsem1, ssem0, ssem1):
    c = lax.axis_index("c")
    s = lax.axis_index("s")
    wid = s * NC + c
    ebase = wid * EPW

    pltpu.sync_copy(col_hbm.at[pl.ds(ebase, EPW)], col_all)

    @pl.when(s < 15)
    def _():
        pltpu.sync_copy(zero_hbm.at[pl.ds(0, SLAB)],
                        acc.at[pl.ds(s * SLAB, SLAB)])

    @pl.when(s == 15)
    def _():
        pltpu.sync_copy(zero_hbm, acc.at[pl.ds(15 * SLAB, LAST)])

    plsc.subcore_barrier()

    def idx_issue(i, rowb, valb, rsem, vsem):
        base = ebase + i * K
        pltpu.async_copy(row_hbm.at[pl.ds(base, K)], rowb, rsem)
        pltpu.async_copy(val_hbm.at[pl.ds(base, K)], valb, vsem)

    def idx_wait(rowb, valb, rsem, vsem):
        pltpu.make_async_copy(row_hbm.at[pl.ds(0, K)], rowb, rsem).wait()
        pltpu.make_async_copy(val_hbm.at[pl.ds(0, K)], valb, vsem).wait()

    def gather_issue(i, buf, gsem):
        pltpu.async_copy(ego_hbm.at[col_all.at[pl.ds(i * K, K)]], buf, gsem)

    def gather_wait(buf, gsem):
        pltpu.make_async_copy(ego_hbm.at[pl.ds(0, K)], buf, gsem).wait()

    def scat_issue(buf, rowb, ssem):
        pltpu.async_copy(buf, acc.at[rowb], ssem, add=True)

    def scat_wait(buf, ssem):
        pltpu.make_async_copy(buf, acc.at[pl.ds(0, K)], ssem).wait()

    def scale(buf, valb):
        def s16(jj, c2):
            off = pl.multiple_of(jj * 16, 16)
            vals16 = valb[pl.ds(off, 16)]
            for l in range(16):
                j = off + l
                v = vals16[l]
                for g in range(NVEC):
                    sl = pl.ds(g * 16, 16)
                    buf[j, sl] = buf[j, sl] * v
            return c2

        lax.fori_loop(0, K // 16, s16, 0)

    idx_issue(0, row0, val0, rsem0, vsem0)
    gather_issue(0, rows0, gsem0)
    idx_issue(1, row1, val1, rsem1, vsem1)
    gather_issue(1, rows1, gsem1)

    def body(t, carry):
        a = 2 * t
        idx_wait(row0, val0, rsem0, vsem0)
        gather_wait(rows0, gsem0)
        scale(rows0, val0)
        scat_issue(rows0, row0, ssem0)
        idx_wait(row1, val1, rsem1, vsem1)
        gather_wait(rows1, gsem1)
        scale(rows1, val1)
        scat_issue(rows1, row1, ssem1)
        scat_wait(rows0, ssem0)
        idx_issue(a + 2, row0, val0, rsem0, vsem0)
        gather_issue(a + 2, rows0, gsem0)
        nb = jnp.minimum(a + 3, NCHUNK - 1)
        scat_wait(rows1, ssem1)
        idx_issue(nb, row1, val1, rsem1, vsem1)
        gather_issue(nb, rows1, gsem1)
        return carry

    lax.fori_loop(0, (NCHUNK - 1) // 2, body, 0)

    idx_wait(row0, val0, rsem0, vsem0)
    gather_wait(rows0, gsem0)
    scale(rows0, val0)
    scat_issue(rows0, row0, ssem0)
    idx_wait(row1, val1, rsem1, vsem1)
    gather_wait(rows1, gsem1)
    scat_wait(rows0, ssem0)
    plsc.subcore_barrier()

    @pl.when(s < 15)
    def _():
        pltpu.sync_copy(acc.at[pl.ds(s * SLAB, SLAB)],
                        out_hbm.at[c, pl.ds(s * SLAB, SLAB)])

    @pl.when(s == 15)
    def _():
        pltpu.sync_copy(acc.at[pl.ds(15 * SLAB, LAST)],
                        out_hbm.at[c, pl.ds(15 * SLAB, LAST)])


def _dense_body(ego_ref, np_ref, wr_ref, re_ref, w1t_ref, w3t_ref, out_ref):
    ego = ego_ref[...]
    neigh = np_ref[0] + np_ref[1]
    wr = wr_ref[...]
    h = jnp.dot(ego, wr, preferred_element_type=jnp.float32)
    t = jnp.dot(neigh, wr, preferred_element_type=jnp.float32)
    score = jnp.sum(t * jnp.tanh(h + re_ref[...]), axis=1, keepdims=True)
    m = jnp.max(score)
    ex = jnp.exp(score - m)
    neigh = neigh * (ex / jnp.sum(ex))
    a = jnp.dot(ego + neigh, w1t_ref[...], preferred_element_type=jnp.float32)
    b = jnp.dot(ego * neigh, w3t_ref[...], preferred_element_type=jnp.float32)
    out_ref[...] = (jnp.where(a >= 0, a, 0.2 * a)
                    + jnp.where(b >= 0, b, 0.2 * b))


def _dense(ego, neigh_parts, wr, re_, w1t, w3t):
    return pl.pallas_call(
        _dense_body,
        out_shape=jax.ShapeDtypeStruct((N_NODES, EMB), jnp.float32),
    )(ego, neigh_parts, wr, re_, w1t, w3t)


NUI = N_USERS + N_ITEMS


def _dense2_body(ent_ref, ego_ref, np_ref, wr_ref, re_ref, w1t_ref, w3t_ref,
                 user_ref, item_ref):
    ego = ego_ref[...]
    neigh = np_ref[0] + np_ref[1]
    wr = wr_ref[...]
    h = jnp.dot(ego, wr, preferred_element_type=jnp.float32)
    t = jnp.dot(neigh, wr, preferred_element_type=jnp.float32)
    score = jnp.sum(t * jnp.tanh(h + re_ref[...]), axis=1, keepdims=True)
    m = jnp.max(score)
    ex = jnp.exp(score - m)
    attn = ex / jnp.sum(ex)
    egon = ego[:NUI]
    neighn = neigh[:NUI] * attn[:NUI]
    a = jnp.dot(egon + neighn, w1t_ref[...], preferred_element_type=jnp.float32)
    b = jnp.dot(egon * neighn, w3t_ref[...], preferred_element_type=jnp.float32)
    res = (jnp.where(a >= 0, a, 0.2 * a)
           + jnp.where(b >= 0, b, 0.2 * b))
    user_ref[:, 0:EMB] = ent_ref[0:N_USERS]
    user_ref[:, EMB:2 * EMB] = ego[0:N_USERS]
    user_ref[:, 2 * EMB:] = res[0:N_USERS]
    item_ref[:, 0:EMB] = ent_ref[N_USERS:NUI]
    item_ref[:, EMB:2 * EMB] = ego[N_USERS:NUI]
    item_ref[:, 2 * EMB:] = res[N_USERS:NUI]


def _dense2(ent_emb, ego, neigh_parts, wr, re_, w1t, w3t):
    return pl.pallas_call(
        _dense2_body,
        out_shape=(jax.ShapeDtypeStruct((N_USERS, 3 * EMB), jnp.float32),
                   jax.ShapeDtypeStruct((N_ITEMS, 3 * EMB), jnp.float32)),
    )(ent_emb, ego, neigh_parts, wr, re_, w1t, w3t)


def kernel(ent_emb, rel_emb, rel_proj, W1, W3, edge_val, edge_row, edge_col):
    wr = rel_proj[0].reshape(EMB, RELD)
    re_ = rel_emb[0].reshape(1, RELD)
    w1t = W1.T
    w3t = W3.T
    zeros = jnp.zeros((LAST, EMB), jnp.float32)
    parts = _spmm(ent_emb, edge_col, edge_row, edge_val, zeros)
    ego1 = _dense(ent_emb, parts, wr, re_, w1t, w3t)
    parts = _spmm(ego1, edge_col, edge_row, edge_val, zeros)
    return _dense2(ent_emb, ego1, parts, wr, re_, w1t, w3t)
